# baseline (speedup 1.0000x reference)
import functools

import jax
import jax.numpy as jnp
from jax.experimental import pallas as pl
from jax.experimental.pallas import tpu as pltpu

P = 8
C = 32
LANES = P * C
W0 = 8
B = 2


def _pack_conv(w, dtype):
    D = w.shape[0]
    S = w.shape[-1] // C
    w7 = w.reshape(D, 3, 3, C, S, C)
    w7 = jnp.transpose(w7, (0, 4, 1, 2, 3, 5))
    pi = jnp.arange(P)[:, None]
    po = jnp.arange(P)[None, :]
    sel = jnp.stack([(pi - po + 1 == t) for t in range(3)]).astype(w.dtype)
    main = jnp.einsum('tpq,dsytcf->dsypcqf', sel, w7)
    main = main.reshape(D, S, 3, P * C, P * C)
    zed = jnp.zeros((D, S, 3, C, P, C), w.dtype)
    left = zed.at[..., 0, :].set(w7[:, :, :, 0]).reshape(D, S, 3, C, P * C)
    right = zed.at[..., P - 1, :].set(w7[:, :, :, 2]).reshape(D, S, 3, C, P * C)
    edge = jnp.concatenate([left, right], axis=-2)
    return main.astype(dtype), edge.astype(dtype)


def _dfag_kernel(x_ref, wm_ref, we_ref, bt_ref,
                 caw1t_ref, cab1_ref, caw2t_ref, cab2t_ref, afold_ref,
                 tm_ref, te_ref, tbt_ref, gamma_ref,
                 o_ref, pad_ref, ec_ref, *, H, W8, CR, n_dfa):
    HG = H * W8
    BHG = B * HG
    HW = H * W8 * P

    pad_ref[:, 0] = jnp.zeros_like(pad_ref[:, 0])
    pad_ref[:, H + 1] = jnp.zeros_like(pad_ref[:, H + 1])
    ec_ref[:, :, W0:W0 + W8, :] = jnp.zeros_like(ec_ref[:, :, W0:W0 + W8, :])

    cdt = wm_ref.dtype

    def conv(x_flat, mats, biases, relus):
        x4 = x_flat.reshape(B, H, W8, LANES)
        pad_ref[:, 1:H + 1, :, :] = x4
        ec_ref[:, 1:H + 1, W0 + 1:W0 + 1 + W8, :C] = x4[..., LANES - C:]
        ec_ref[:, 1:H + 1, W0 - 1:W0 - 1 + W8, C:] = x4[..., :C]
        nout = len(mats)
        accs = [None] * nout
        for ky in range(3):
            pm = pad_ref[:, ky:ky + H, :, :].reshape(BHG, LANES)
            pm = pm.astype(cdt)
            pe = ec_ref[:, ky:ky + H, W0:W0 + W8, :].reshape(BHG, 2 * C)
            pe = pe.astype(cdt)
            for t, (m3, e3) in enumerate(mats):
                a = jnp.dot(pm, m3[ky], preferred_element_type=jnp.float32)
                a = a + jnp.dot(pe, e3[ky],
                                preferred_element_type=jnp.float32)
                accs[t] = a if accs[t] is None else accs[t] + a
        outs = []
        for acc, bias, relu in zip(accs, biases, relus):
            if bias is not None:
                acc = acc + bias
            if relu:
                acc = jnp.maximum(acc, 0.0)
            outs.append(acc)
        return outs

    def dfa_mats(d, s):
        return ([wm_ref[d, s, ky] for ky in range(3)],
                [we_ref[d, s, ky] for ky in range(3)])

    def fold32(v, op):
        v = op(v[..., :128], v[..., 128:])
        v = op(v[..., :64], v[..., 64:])
        return op(v[..., :32], v[..., 32:])

    def tile8(v):
        return jnp.concatenate([v] * P, axis=-1)

    def ca_layer(x_flat, d, r):
        xb = x_flat.reshape(B, HG, LANES)
        pooled = jnp.sum(xb, axis=1) * (1.0 / HW)
        w1 = caw1t_ref[d][:, r * CR:(r + 1) * CR]
        b1 = cab1_ref[d][:, r * CR:(r + 1) * CR]
        h = jnp.maximum(
            jnp.dot(pooled, w1, preferred_element_type=jnp.float32) + b1, 0.0)
        w2v = caw2t_ref[d, r]
        acc = cab2t_ref[d, r]
        for j in range(CR):
            acc = acc + h[:, j:j + 1] * w2v[j:j + 1, :]
        y = jax.nn.sigmoid(acc)
        return (xb * y[:, None, :]).reshape(BHG, LANES)

    def rcab(x_flat, d, r):
        s1, s2 = 2 * r, 2 * r + 1
        (h1,) = conv(x_flat, [dfa_mats(d, s1)], [bt_ref[d, s1]], [True])
        (h2,) = conv(h1, [dfa_mats(d, s2)], [bt_ref[d, s2]], [False])
        return ca_layer(h2, d, r) + x_flat

    def dfa_block(d, x_flat):
        x_flat = rcab(x_flat, d, 0)
        x_flat = rcab(x_flat, d, 1)
        q, k = conv(x_flat, [dfa_mats(d, 4), dfa_mats(d, 5)],
                    [None, None], [False, False])
        (v,) = conv(k, [dfa_mats(d, 6)], [None], [False])
        e4 = (q * k).reshape(B, H, W8, LANES)
        m = fold32(jnp.max(e4, axis=2), jnp.maximum)
        e = jnp.exp(e4 - tile8(m)[:, :, None, :])
        s = jnp.dot(jnp.sum(e, axis=2).reshape(B * H, LANES), afold_ref[...],
                    preferred_element_type=jnp.float32)
        s = s.reshape(B, H, LANES)
        attn = e * pl.reciprocal(s, approx=False)[:, :, None, :]
        out = gamma_ref[d] * (v.reshape(B, H, W8, LANES) * attn)
        return out.reshape(BHG, LANES)

    x0 = x_ref[0].reshape(BHG, LANES)
    res = jax.lax.fori_loop(0, n_dfa, dfa_block, x0)
    tmats = ([tm_ref[0, 0, ky] for ky in range(3)],
             [te_ref[0, 0, ky] for ky in range(3)])
    (tail,) = conv(res, [tmats], [tbt_ref[...]], [False])
    o_ref[0] = (tail + x0).reshape(B, H, W8, LANES).astype(o_ref.dtype)


def kernel(x, w, b, caw1, cab1, caw2, cab2, tw, tb, gamma):
    N, H, W, C_ = x.shape
    assert C_ == C and W % P == 0 and N % B == 0
    W8 = W // P
    n_dfa = w.shape[0]
    CR = caw1.shape[-1] // 2
    cdt = jnp.float32

    wm, we = _pack_conv(w, cdt)
    tm, te = _pack_conv(tw[None, ...], cdt)
    bt = jnp.tile(b.reshape(n_dfa, 7, 1, 1, C)[:, :4], (1, 1, 1, P, 1))
    bt = bt.reshape(n_dfa, 4, 1, LANES)
    tbt = jnp.tile(tb, (1, P))
    caw1t = jnp.tile(caw1, (1, P, 1))
    caw2t = jnp.tile(caw2.reshape(n_dfa, CR, 2, 1, C), (1, 1, 1, P, 1))
    caw2t = caw2t.reshape(n_dfa, CR, 2, LANES).transpose(0, 2, 1, 3)
    cab2t = jnp.tile(cab2.reshape(n_dfa, 2, 1, 1, C), (1, 1, 1, P, 1))
    cab2t = cab2t.reshape(n_dfa, 2, 1, LANES)
    afold = jnp.tile(jnp.eye(C, dtype=jnp.float32), (P, P))

    xp = x.reshape(N // B, B, H, W8, LANES)

    def full_spec(a):
        nd = a.ndim
        return pl.BlockSpec(a.shape, lambda n: (0,) * nd)

    _body = functools.partial(_dfag_kernel, H=H, W8=W8, CR=CR, n_dfa=n_dfa)

    out = pl.pallas_call(
        _body,
        out_shape=jax.ShapeDtypeStruct((N // B, B, H, W8, LANES), x.dtype),
        grid=(N // B,),
        in_specs=[
            pl.BlockSpec((1, B, H, W8, LANES), lambda n: (n, 0, 0, 0, 0)),
            full_spec(wm), full_spec(we), full_spec(bt),
            full_spec(caw1t), full_spec(cab1), full_spec(caw2t),
            full_spec(cab2t), full_spec(afold),
            full_spec(tm), full_spec(te), full_spec(tbt),
            pl.BlockSpec(memory_space=pltpu.MemorySpace.SMEM),
        ],
        out_specs=pl.BlockSpec((1, B, H, W8, LANES),
                               lambda n: (n, 0, 0, 0, 0)),
        scratch_shapes=[pltpu.VMEM((B, H + 2, W8, LANES), jnp.float32),
                        pltpu.VMEM((B, H + 2, 24, 2 * C), jnp.float32)],
        compiler_params=pltpu.CompilerParams(
            dimension_semantics=("parallel",)),
    )(xp, wm, we, bt, caw1t, cab1, caw2t, cab2t, afold, tm, te, tbt, gamma)
    return out.reshape(N, H, W, C)


# speedup vs baseline: 1.0673x; 1.0673x over previous
import functools

import jax
import jax.numpy as jnp
from jax.experimental import pallas as pl
from jax.experimental.pallas import tpu as pltpu

P = 8
C = 32
LANES = P * C
W0 = 8
B = 4


def _pack_conv(w, dtype):
    D = w.shape[0]
    S = w.shape[-1] // C
    w7 = w.reshape(D, 3, 3, C, S, C)
    w7 = jnp.transpose(w7, (0, 4, 1, 2, 3, 5))
    pi = jnp.arange(P)[:, None]
    po = jnp.arange(P)[None, :]
    sel = jnp.stack([(pi - po + 1 == t) for t in range(3)]).astype(w.dtype)
    main = jnp.einsum('tpq,dsytcf->dsypcqf', sel, w7)
    main = main.reshape(D, S, 3, P * C, P * C)
    zed = jnp.zeros((D, S, 3, C, P, C), w.dtype)
    left = zed.at[..., 0, :].set(w7[:, :, :, 0]).reshape(D, S, 3, C, P * C)
    right = zed.at[..., P - 1, :].set(w7[:, :, :, 2]).reshape(D, S, 3, C, P * C)
    edge = jnp.concatenate([left, right], axis=-2)
    return main.astype(dtype), edge.astype(dtype)


def _dfag_kernel(x_ref, wm_ref, we_ref, bt_ref,
                 caw1t_ref, cab1_ref, caw2t_ref, cab2t_ref, afold_ref,
                 tm_ref, te_ref, tbt_ref, gamma_ref,
                 o_ref, pad_ref, ec_ref, *, H, W8, CR, n_dfa):
    HG = H * W8
    BHG = B * HG
    HW = H * W8 * P

    pad_ref[:, 0] = jnp.zeros_like(pad_ref[:, 0])
    pad_ref[:, H + 1] = jnp.zeros_like(pad_ref[:, H + 1])
    ec_ref[:, :, W0:W0 + W8, :] = jnp.zeros_like(ec_ref[:, :, W0:W0 + W8, :])

    cdt = wm_ref.dtype

    def conv(x_flat, mats, biases, relus):
        x4 = x_flat.reshape(B, H, W8, LANES)
        pad_ref[:, 1:H + 1, :, :] = x4
        ec_ref[:, 1:H + 1, W0 + 1:W0 + 1 + W8, :C] = x4[..., LANES - C:]
        ec_ref[:, 1:H + 1, W0 - 1:W0 - 1 + W8, C:] = x4[..., :C]
        nout = len(mats)
        accs = [None] * nout
        for ky in range(3):
            pm = pad_ref[:, ky:ky + H, :, :].reshape(BHG, LANES)
            pm = pm.astype(cdt)
            pe = ec_ref[:, ky:ky + H, W0:W0 + W8, :].reshape(BHG, 2 * C)
            pe = pe.astype(cdt)
            for t, (m3, e3) in enumerate(mats):
                a = jnp.dot(pm, m3[ky], preferred_element_type=jnp.float32)
                a = a + jnp.dot(pe, e3[ky],
                                preferred_element_type=jnp.float32)
                accs[t] = a if accs[t] is None else accs[t] + a
        outs = []
        for acc, bias, relu in zip(accs, biases, relus):
            if bias is not None:
                acc = acc + bias
            if relu:
                acc = jnp.maximum(acc, 0.0)
            outs.append(acc)
        return outs

    def dfa_mats(d, s):
        return ([wm_ref[d, s, ky] for ky in range(3)],
                [we_ref[d, s, ky] for ky in range(3)])

    def fold32(v, op):
        v = op(v[..., :128], v[..., 128:])
        v = op(v[..., :64], v[..., 64:])
        return op(v[..., :32], v[..., 32:])

    def tile8(v):
        return jnp.concatenate([v] * P, axis=-1)

    def ca_layer(x_flat, d, r):
        xb = x_flat.reshape(B, HG, LANES)
        pooled = jnp.sum(xb, axis=1) * (1.0 / HW)
        w1 = caw1t_ref[d][:, r * CR:(r + 1) * CR]
        b1 = cab1_ref[d][:, r * CR:(r + 1) * CR]
        h = jnp.maximum(
            jnp.dot(pooled, w1, preferred_element_type=jnp.float32) + b1, 0.0)
        w2v = caw2t_ref[d, r]
        acc = cab2t_ref[d, r]
        for j in range(CR):
            acc = acc + h[:, j:j + 1] * w2v[j:j + 1, :]
        y = jax.nn.sigmoid(acc)
        return (xb * y[:, None, :]).reshape(BHG, LANES)

    def rcab(x_flat, d, r):
        s1, s2 = 2 * r, 2 * r + 1
        (h1,) = conv(x_flat, [dfa_mats(d, s1)], [bt_ref[d, s1]], [True])
        (h2,) = conv(h1, [dfa_mats(d, s2)], [bt_ref[d, s2]], [False])
        return ca_layer(h2, d, r) + x_flat

    def dfa_block(d, x_flat):
        x_flat = rcab(x_flat, d, 0)
        x_flat = rcab(x_flat, d, 1)
        q, k = conv(x_flat, [dfa_mats(d, 4), dfa_mats(d, 5)],
                    [None, None], [False, False])
        (v,) = conv(k, [dfa_mats(d, 6)], [None], [False])
        e4 = (q * k).reshape(B, H, W8, LANES)
        m = fold32(jnp.max(e4, axis=2), jnp.maximum)
        e = jnp.exp(e4 - tile8(m)[:, :, None, :])
        s = jnp.dot(jnp.sum(e, axis=2).reshape(B * H, LANES), afold_ref[...],
                    preferred_element_type=jnp.float32)
        s = s.reshape(B, H, LANES)
        attn = e * pl.reciprocal(s, approx=False)[:, :, None, :]
        out = gamma_ref[d] * (v.reshape(B, H, W8, LANES) * attn)
        return out.reshape(BHG, LANES)

    x0 = x_ref[0].reshape(BHG, LANES)
    res = jax.lax.fori_loop(0, n_dfa, dfa_block, x0)
    tmats = ([tm_ref[0, 0, ky] for ky in range(3)],
             [te_ref[0, 0, ky] for ky in range(3)])
    (tail,) = conv(res, [tmats], [tbt_ref[...]], [False])
    o_ref[0] = (tail + x0).reshape(B, H, W8, LANES).astype(o_ref.dtype)


def kernel(x, w, b, caw1, cab1, caw2, cab2, tw, tb, gamma):
    N, H, W, C_ = x.shape
    assert C_ == C and W % P == 0 and N % B == 0
    W8 = W // P
    n_dfa = w.shape[0]
    CR = caw1.shape[-1] // 2
    cdt = jnp.bfloat16

    wm, we = _pack_conv(w, cdt)
    tm, te = _pack_conv(tw[None, ...], cdt)
    bt = jnp.tile(b.reshape(n_dfa, 7, 1, 1, C)[:, :4], (1, 1, 1, P, 1))
    bt = bt.reshape(n_dfa, 4, 1, LANES)
    tbt = jnp.tile(tb, (1, P))
    caw1t = jnp.tile(caw1, (1, P, 1))
    caw2t = jnp.tile(caw2.reshape(n_dfa, CR, 2, 1, C), (1, 1, 1, P, 1))
    caw2t = caw2t.reshape(n_dfa, CR, 2, LANES).transpose(0, 2, 1, 3)
    cab2t = jnp.tile(cab2.reshape(n_dfa, 2, 1, 1, C), (1, 1, 1, P, 1))
    cab2t = cab2t.reshape(n_dfa, 2, 1, LANES)
    afold = jnp.tile(jnp.eye(C, dtype=jnp.float32), (P, P))

    xp = x.reshape(N // B, B, H, W8, LANES)

    def full_spec(a):
        nd = a.ndim
        return pl.BlockSpec(a.shape, lambda n: (0,) * nd)

    _body = functools.partial(_dfag_kernel, H=H, W8=W8, CR=CR, n_dfa=n_dfa)

    out = pl.pallas_call(
        _body,
        out_shape=jax.ShapeDtypeStruct((N // B, B, H, W8, LANES), x.dtype),
        grid=(N // B,),
        in_specs=[
            pl.BlockSpec((1, B, H, W8, LANES), lambda n: (n, 0, 0, 0, 0)),
            full_spec(wm), full_spec(we), full_spec(bt),
            full_spec(caw1t), full_spec(cab1), full_spec(caw2t),
            full_spec(cab2t), full_spec(afold),
            full_spec(tm), full_spec(te), full_spec(tbt),
            pl.BlockSpec(memory_space=pltpu.MemorySpace.SMEM),
        ],
        out_specs=pl.BlockSpec((1, B, H, W8, LANES),
                               lambda n: (n, 0, 0, 0, 0)),
        scratch_shapes=[pltpu.VMEM((B, H + 2, W8, LANES), jnp.float32),
                        pltpu.VMEM((B, H + 2, 24, 2 * C), jnp.float32)],
        compiler_params=pltpu.CompilerParams(
            dimension_semantics=("parallel",)),
    )(xp, wm, we, bt, caw1t, cab1, caw2t, cab2t, afold, tm, te, tbt, gamma)
    return out.reshape(N, H, W, C)
